# Initial kernel scaffold; baseline (speedup 1.0000x reference)
import jax
import jax.numpy as jnp
from jax.experimental import pallas as pl


def kernel(x, logdet, conditioning, W1, b1, W2, b2, W3, b3):
    raise NotImplementedError("write your pallas kernel here")


# speedup vs baseline: 271.2560x; 271.2560x over previous
import numpy as np
import jax
import jax.numpy as jnp
from jax.experimental import pallas as pl
from jax.experimental.pallas import tpu as pltpu

_K = 8
_TAIL = 3.0
_MBW = 1e-3
_MBH = 1e-3
_MD = 1e-3
_B, _C, _H, _W = 16, 12, 128, 128
_COND = _C // 3
_HID = 64
_OUTP = 288
_TR = 32
_NT = _H // _TR
_PT = _TR * _W
_PH = (_TR + 2) * _W
_PZ = (_TR + 4) * _W
_DCONST = float(np.log(np.expm1(1.0 - _MD)))


def _shift_cols(v, dx, col):
    if dx == 1:
        return v
    if dx == 0:
        u = pltpu.roll(v, 1, axis=1)
        return jnp.where(col == 0, 0.0, u)
    u = pltpu.roll(v, v.shape[1] - 1, axis=1)
    return jnp.where(col == _W - 1, 0.0, u)


def _conv3x3(src, wr, nout_lanes):
    cin, nl = src.shape
    col = jax.lax.broadcasted_iota(jnp.int32, (1, nl), 1) % _W
    acc = None
    for dx in range(3):
        s = _shift_cols(src, dx, col)
        for dy in range(3):
            tap = jax.lax.dot(
                wr[dy, dx], s[:, dy * _W:dy * _W + nout_lanes],
                preferred_element_type=jnp.float32)
            acc = tap if acc is None else acc + tap
    return acc


def _shift_bins(v, k):
    z = jnp.zeros((v.shape[0], k, v.shape[2]), v.dtype)
    return jnp.concatenate([z, v[:, :v.shape[1] - k, :]], axis=1)


def _bin_edges(u):
    m = jnp.max(u, axis=1, keepdims=True)
    e = jnp.exp(u - m)
    w = _MBW + (1.0 - _MBW * _K) * (e / jnp.sum(e, axis=1, keepdims=True))
    c = w
    c = c + _shift_bins(c, 1)
    c = c + _shift_bins(c, 2)
    c = c + _shift_bins(c, 4)
    cw = -_TAIL + 2.0 * _TAIL * c
    top = jnp.full((u.shape[0], 1, u.shape[2]), -_TAIL, u.dtype)
    cw9 = jnp.concatenate([top, cw], axis=1)
    kidx = jax.lax.broadcasted_iota(jnp.int32, cw9.shape, 1)
    cw9 = jnp.where(kidx == _K, _TAIL, cw9)
    return cw9, cw9[:, 1:, :] - cw9[:, :-1, :]


def _kernel(xp_ref, cp_ref, w1_ref, b1_ref, w2_ref, b2_ref, w3_ref, b3_ref,
            xout_ref, lad_ref):
    t = pl.program_id(1)
    lane_z = jax.lax.broadcasted_iota(jnp.int32, (1, _PZ), 1)
    frozen_z = ((lane_z // _W + lane_z) % 2).astype(jnp.float32)

    xz = xp_ref[:, pl.ds(t * _PT, _PZ)]
    cz = cp_ref[:, pl.ds(t * _PT, _PZ)]
    z = jnp.concatenate([xz * frozen_z, cz], axis=0)

    h1 = jnp.maximum(_conv3x3(z, w1_ref[...], _PH) + b1_ref[...], 0.0)
    h2 = jnp.maximum(
        jax.lax.dot(w2_ref[...], h1, preferred_element_type=jnp.float32)
        + b2_ref[...], 0.0)
    lane_h = jax.lax.broadcasted_iota(jnp.int32, (1, _PH), 1)
    grow = t * _TR - 1 + lane_h // _W
    h2 = jnp.where((grow >= 0) & (grow < _H), h2, 0.0)
    params = _conv3x3(h2, w3_ref[...], _PT) + b3_ref[...]

    uw = params[0:96].reshape(_C, _K, _PT)
    uh = params[96:192].reshape(_C, _K, _PT)
    ud8 = params[192:288].reshape(_C, _K, _PT)

    kidx = jax.lax.broadcasted_iota(jnp.int32, (_C, _K, _PT), 1)
    d1u = jnp.where(kidx == _K - 1, _DCONST, ud8)
    d0u = jnp.where(kidx == 0, _DCONST, _shift_bins(ud8, 1))
    d0 = _MD + jax.nn.softplus(d0u)
    d1 = _MD + jax.nn.softplus(d1u)

    cw9, w_eff = _bin_edges(uw)
    ch9, h_eff = _bin_edges(uh)

    xraw = xz[:, 2 * _W:2 * _W + _PT]
    lane_o = jax.lax.broadcasted_iota(jnp.int32, (1, _PT), 1)
    frozen_o = ((lane_o // _W + lane_o) % 2).astype(jnp.float32)
    xa = xraw * (1.0 - frozen_o)
    xc = jnp.clip(xa, -_TAIL, _TAIL)

    cmp = (xc[:, None, :] >= cw9).astype(jnp.int32)
    idx = jnp.clip(jnp.sum(cmp, axis=1) - 1, 0, _K - 1)
    oh = (idx[:, None, :] == kidx).astype(jnp.float32)

    def gsel(tbl):
        return jnp.sum(oh * tbl, axis=1)

    icw = gsel(cw9[:, :_K, :])
    iw = gsel(w_eff)
    ich = gsel(ch9[:, :_K, :])
    ih = gsel(h_eff)
    id0 = gsel(d0)
    id1 = gsel(d1)
    idl = ih / iw

    theta = (xc - icw) / iw
    t1 = theta * (1.0 - theta)
    th2 = theta * theta
    num = ih * (idl * th2 + id0 * t1)
    den = idl + (id0 + id1 - 2.0 * idl) * t1
    y = ich + num / den
    omt = 1.0 - theta
    dnum = idl * idl * (id1 * th2 + 2.0 * idl * t1 + id0 * omt * omt)
    lad = jnp.log(dnum) - 2.0 * jnp.log(den)

    inside = (xa >= -_TAIL) & (xa <= _TAIL)
    yout = jnp.where(inside, y, xa)
    ladout = jnp.where(inside, lad, 0.0)

    xout_ref[...] = xraw * frozen_o + yout
    lad_ref[...] = jnp.sum(ladout, axis=0, keepdims=True)


def kernel(x, logdet, conditioning, W1, b1, W2, b2, W3, b3):
    xp = jnp.pad(x, ((0, 0), (0, 0), (2, 2), (0, 0))).reshape(_B, _C, (_H + 4) * _W)
    cp = jnp.pad(conditioning, ((0, 0), (0, 0), (2, 2), (0, 0)))
    cp = cp.reshape(_B, _COND, (_H + 4) * _W)

    w1r = jnp.transpose(W1, (2, 3, 0, 1))
    b1r = b1.reshape(_HID, 1)
    w2m = W2[:, :, 0, 0]
    b2r = b2.reshape(_HID, 1)

    perm = np.zeros(_OUTP, np.int32)
    step = 3 * _K - 1
    for c in range(_C):
        for k in range(_K):
            perm[c * _K + k] = c * step + k
            perm[96 + c * _K + k] = c * step + _K + k
        for k in range(7):
            perm[192 + c * _K + k] = c * step + 2 * _K + k
        perm[192 + c * _K + 7] = _C * step
    w3p = jnp.concatenate([W3, jnp.zeros((1,) + W3.shape[1:], W3.dtype)], 0)[perm]
    b3p = jnp.concatenate([b3, jnp.zeros((1,), b3.dtype)], 0)[perm]
    w3r = jnp.transpose(w3p, (2, 3, 0, 1))
    b3r = b3p.reshape(_OUTP, 1)

    xout, ladp = pl.pallas_call(
        _kernel,
        grid=(_B, _NT),
        in_specs=[
            pl.BlockSpec((None, _C, (_H + 4) * _W), lambda b, t: (b, 0, 0)),
            pl.BlockSpec((None, _COND, (_H + 4) * _W), lambda b, t: (b, 0, 0)),
            pl.BlockSpec((3, 3, _HID, _C + _COND), lambda b, t: (0, 0, 0, 0)),
            pl.BlockSpec((_HID, 1), lambda b, t: (0, 0)),
            pl.BlockSpec((_HID, _HID), lambda b, t: (0, 0)),
            pl.BlockSpec((_HID, 1), lambda b, t: (0, 0)),
            pl.BlockSpec((3, 3, _OUTP, _HID), lambda b, t: (0, 0, 0, 0)),
            pl.BlockSpec((_OUTP, 1), lambda b, t: (0, 0)),
        ],
        out_specs=[
            pl.BlockSpec((None, _C, _PT), lambda b, t: (b, 0, t)),
            pl.BlockSpec((None, None, 1, _PT), lambda b, t: (b, t, 0, 0)),
        ],
        out_shape=[
            jax.ShapeDtypeStruct((_B, _C, _H * _W), jnp.float32),
            jax.ShapeDtypeStruct((_B, _NT, 1, _PT), jnp.float32),
        ],
        compiler_params=pltpu.CompilerParams(
            dimension_semantics=("parallel", "arbitrary")),
    )(xp, cp, w1r, b1r, w2m, b2r, w3r, b3r)

    x_out = xout.reshape(_B, _C, _H, _W)
    logdet_out = logdet + jnp.sum(ladp, axis=(1, 2, 3))
    return x_out, logdet_out


# speedup vs baseline: 306.6774x; 1.1306x over previous
import numpy as np
import jax
import jax.numpy as jnp
from jax.experimental import pallas as pl
from jax.experimental.pallas import tpu as pltpu

_K = 8
_TAIL = 3.0
_MBW = 1e-3
_MBH = 1e-3
_MD = 1e-3
_B, _C, _H, _W = 16, 12, 128, 128
_COND = _C // 3
_HID = 64
_OUTP = 288
_TR = 32
_NT = _H // _TR
_PT = _TR * _W
_PH = (_TR + 2) * _W
_PZ = (_TR + 4) * _W
_DCONST = float(np.log(np.expm1(1.0 - _MD)))


def _shift_cols(v, dx, col):
    if dx == 1:
        return v
    if dx == 0:
        u = pltpu.roll(v, 1, axis=1)
        return jnp.where(col == 0, 0.0, u)
    u = pltpu.roll(v, v.shape[1] - 1, axis=1)
    return jnp.where(col == _W - 1, 0.0, u)


def _conv3x3(src, wr, nout_lanes):
    cin, nl = src.shape
    col = jax.lax.broadcasted_iota(jnp.int32, (1, nl), 1) % _W
    acc = None
    for dx in range(3):
        s = _shift_cols(src, dx, col)
        for dy in range(3):
            tap = jax.lax.dot(
                wr[dy, dx], s[:, dy * _W:dy * _W + nout_lanes],
                preferred_element_type=jnp.float32)
            acc = tap if acc is None else acc + tap
    return acc


def _norm_bins(u, mb):
    m = jnp.max(u, axis=1, keepdims=True)
    e = jnp.exp(u - m)
    return mb + (1.0 - mb * _K) * (e / jnp.sum(e, axis=1, keepdims=True))


def _kernel(xp_ref, cp_ref, w1_ref, b1_ref, w2_ref, b2_ref, w3_ref, b3_ref,
            xout_ref, lad_ref):
    t = pl.program_id(1)
    lane_z = jax.lax.broadcasted_iota(jnp.int32, (1, _PZ), 1)
    frozen_z = ((lane_z // _W + lane_z) % 2).astype(jnp.float32)

    xz = xp_ref[:, pl.ds(t * _PT, _PZ)]
    cz = cp_ref[:, pl.ds(t * _PT, _PZ)]
    z = jnp.concatenate([xz * frozen_z, cz], axis=0)

    h1 = jnp.maximum(_conv3x3(z, w1_ref[...], _PH) + b1_ref[...], 0.0)
    h2 = jnp.maximum(
        jax.lax.dot(w2_ref[...], h1, preferred_element_type=jnp.float32)
        + b2_ref[...], 0.0)
    lane_h = jax.lax.broadcasted_iota(jnp.int32, (1, _PH), 1)
    grow = t * _TR - 1 + lane_h // _W
    h2 = jnp.where((grow >= 0) & (grow < _H), h2, 0.0)
    params = _conv3x3(h2, w3_ref[...], _PT) + b3_ref[...]

    uw = params[0:96].reshape(_C, _K, _PT)
    uh = params[96:192].reshape(_C, _K, _PT)
    ud8 = params[192:288].reshape(_C, _K, _PT)

    wh = jnp.concatenate([_norm_bins(uw, _MBW).reshape(96, _PT),
                          _norm_bins(uh, _MBH).reshape(96, _PT)], axis=0)
    ri = jax.lax.broadcasted_iota(jnp.int32, (192, 192), 0)
    ci = jax.lax.broadcasted_iota(jnp.int32, (192, 192), 1)
    tri = ((ri >= ci) & (ri // _K == ci // _K)).astype(jnp.float32)
    cs = jax.lax.dot(tri, wh, preferred_element_type=jnp.float32)
    edges = -_TAIL + 2.0 * _TAIL * cs
    lastrow = (jax.lax.broadcasted_iota(jnp.int32, (192, 1), 0) % _K) == _K - 1
    edges = jnp.where(lastrow, _TAIL, edges)
    cw8 = edges[0:96].reshape(_C, _K, _PT)
    ch8 = edges[96:192].reshape(_C, _K, _PT)

    kidx = jax.lax.broadcasted_iota(jnp.int32, (_C, _K, _PT), 1)
    dv = _MD + jax.nn.softplus(jnp.where(kidx == _K - 1, _DCONST, ud8))

    xraw = xz[:, 2 * _W:2 * _W + _PT]
    lane_o = jax.lax.broadcasted_iota(jnp.int32, (1, _PT), 1)
    frozen_o = ((lane_o // _W + lane_o) % 2).astype(jnp.float32)
    xa = xraw * (1.0 - frozen_o)
    xc = jnp.clip(xa, -_TAIL, _TAIL)

    cmp = (xc[:, None, :] >= cw8).astype(jnp.int32)
    idx = jnp.clip(jnp.sum(cmp, axis=1), 0, _K - 1)
    ohr = (idx[:, None, :] == kidx).astype(jnp.float32)
    ohl = (idx[:, None, :] == kidx + 1).astype(jnp.float32)
    zat0 = (idx == 0)

    def gsel(tbl, o):
        return jnp.sum(o * tbl, axis=1)

    icw = gsel(cw8, ohl) + jnp.where(zat0, -_TAIL, 0.0)
    iw = gsel(cw8, ohr) - icw
    ich = gsel(ch8, ohl) + jnp.where(zat0, -_TAIL, 0.0)
    ih = gsel(ch8, ohr) - ich
    id1 = gsel(dv, ohr)
    id0 = gsel(dv, ohl) + jnp.where(zat0, 1.0, 0.0)
    idl = ih / iw

    theta = (xc - icw) / iw
    t1 = theta * (1.0 - theta)
    th2 = theta * theta
    num = ih * (idl * th2 + id0 * t1)
    den = idl + (id0 + id1 - 2.0 * idl) * t1
    y = ich + num / den
    omt = 1.0 - theta
    dnum = idl * idl * (id1 * th2 + 2.0 * idl * t1 + id0 * omt * omt)
    lad = jnp.log(dnum) - 2.0 * jnp.log(den)

    inside = (xa >= -_TAIL) & (xa <= _TAIL)
    yout = jnp.where(inside, y, xa)
    ladout = jnp.where(inside, lad, 0.0)

    xout_ref[...] = xraw * frozen_o + yout
    lad_ref[...] = jnp.sum(ladout, axis=0, keepdims=True)


def kernel(x, logdet, conditioning, W1, b1, W2, b2, W3, b3):
    xp = jnp.pad(x, ((0, 0), (0, 0), (2, 2), (0, 0))).reshape(_B, _C, (_H + 4) * _W)
    cp = jnp.pad(conditioning, ((0, 0), (0, 0), (2, 2), (0, 0)))
    cp = cp.reshape(_B, _COND, (_H + 4) * _W)

    w1r = jnp.transpose(W1, (2, 3, 0, 1))
    b1r = b1.reshape(_HID, 1)
    w2m = W2[:, :, 0, 0]
    b2r = b2.reshape(_HID, 1)

    perm = np.zeros(_OUTP, np.int32)
    step = 3 * _K - 1
    for c in range(_C):
        for k in range(_K):
            perm[c * _K + k] = c * step + k
            perm[96 + c * _K + k] = c * step + _K + k
        for k in range(7):
            perm[192 + c * _K + k] = c * step + 2 * _K + k
        perm[192 + c * _K + 7] = _C * step
    w3p = jnp.concatenate([W3, jnp.zeros((1,) + W3.shape[1:], W3.dtype)], 0)[perm]
    b3p = jnp.concatenate([b3, jnp.zeros((1,), b3.dtype)], 0)[perm]
    w3r = jnp.transpose(w3p, (2, 3, 0, 1))
    b3r = b3p.reshape(_OUTP, 1)

    xout, ladp = pl.pallas_call(
        _kernel,
        grid=(_B, _NT),
        in_specs=[
            pl.BlockSpec((None, _C, (_H + 4) * _W), lambda b, t: (b, 0, 0)),
            pl.BlockSpec((None, _COND, (_H + 4) * _W), lambda b, t: (b, 0, 0)),
            pl.BlockSpec((3, 3, _HID, _C + _COND), lambda b, t: (0, 0, 0, 0)),
            pl.BlockSpec((_HID, 1), lambda b, t: (0, 0)),
            pl.BlockSpec((_HID, _HID), lambda b, t: (0, 0)),
            pl.BlockSpec((_HID, 1), lambda b, t: (0, 0)),
            pl.BlockSpec((3, 3, _OUTP, _HID), lambda b, t: (0, 0, 0, 0)),
            pl.BlockSpec((_OUTP, 1), lambda b, t: (0, 0)),
        ],
        out_specs=[
            pl.BlockSpec((None, _C, _PT), lambda b, t: (b, 0, t)),
            pl.BlockSpec((None, None, 1, _PT), lambda b, t: (b, t, 0, 0)),
        ],
        out_shape=[
            jax.ShapeDtypeStruct((_B, _C, _H * _W), jnp.float32),
            jax.ShapeDtypeStruct((_B, _NT, 1, _PT), jnp.float32),
        ],
        compiler_params=pltpu.CompilerParams(
            dimension_semantics=("parallel", "arbitrary")),
    )(xp, cp, w1r, b1r, w2m, b2r, w3r, b3r)

    x_out = xout.reshape(_B, _C, _H, _W)
    logdet_out = logdet + jnp.sum(ladp, axis=(1, 2, 3))
    return x_out, logdet_out


# speedup vs baseline: 412.2779x; 1.3443x over previous
import numpy as np
import jax
import jax.numpy as jnp
from jax.experimental import pallas as pl
from jax.experimental.pallas import tpu as pltpu

_K = 8
_TAIL = 3.0
_MBW = 1e-3
_MBH = 1e-3
_MD = 1e-3
_B, _C, _H, _W = 16, 12, 128, 128
_COND = _C // 3
_HID = 64
_OUTP = 288
_TR = 32
_NT = _H // _TR
_PT = _TR * _W
_PH = (_TR + 2) * _W
_PZ = (_TR + 4) * _W
_DCONST = float(np.log(np.expm1(1.0 - _MD)))


def _shift_cols(v, dx, col):
    if dx == 1:
        return v
    if dx == 0:
        u = pltpu.roll(v, 1, axis=1)
        return jnp.where(col == 0, 0.0, u)
    u = pltpu.roll(v, v.shape[1] - 1, axis=1)
    return jnp.where(col == _W - 1, 0.0, u)


def _conv3x3(src, wr, nout_lanes):
    cin, nl = src.shape
    col = jax.lax.broadcasted_iota(jnp.int32, (1, nl), 1) % _W
    acc = None
    for dx in range(3):
        s = _shift_cols(src, dx, col)
        for dy in range(3):
            tap = jax.lax.dot(
                wr[dy, dx], s[:, dy * _W:dy * _W + nout_lanes],
                preferred_element_type=jnp.float32)
            acc = tap if acc is None else acc + tap
    return acc


def _iota2(shape, dim):
    return jax.lax.broadcasted_iota(jnp.int32, shape, dim)


def _kernel(xp_ref, cp_ref, w1_ref, b1_ref, w2_ref, b2_ref, w3_ref, b3_ref,
            xout_ref, lad_ref):
    t = pl.program_id(1)
    lane_z = jax.lax.broadcasted_iota(jnp.int32, (1, _PZ), 1)
    frozen_z = ((lane_z // _W + lane_z) % 2).astype(jnp.float32)

    xz = xp_ref[:, pl.ds(t * _PT, _PZ)]
    cz = cp_ref[:, pl.ds(t * _PT, _PZ)]
    z = jnp.concatenate([xz * frozen_z, cz], axis=0)

    h1 = jnp.maximum(_conv3x3(z, w1_ref[...], _PH) + b1_ref[...], 0.0)
    h2 = jnp.maximum(
        jax.lax.dot(w2_ref[...], h1, preferred_element_type=jnp.float32)
        + b2_ref[...], 0.0)
    lane_h = jax.lax.broadcasted_iota(jnp.int32, (1, _PH), 1)
    grow = t * _TR - 1 + lane_h // _W
    h2 = jnp.where((grow >= 0) & (grow < _H), h2, 0.0)
    params = _conv3x3(h2, w3_ref[...], _PT) + b3_ref[...]

    uw = params[0:96].reshape(_C, _K, _PT)
    uh = params[96:192].reshape(_C, _K, _PT)
    ud8 = params[192:288].reshape(_C, _K, _PT)

    mw = jnp.max(uw, axis=1, keepdims=True)
    mh = jnp.max(uh, axis=1, keepdims=True)
    e192 = jnp.concatenate([jnp.exp(uw - mw).reshape(96, _PT),
                            jnp.exp(uh - mh).reshape(96, _PT)], axis=0)
    ri = _iota2((192, 192), 0)
    ci = _iota2((192, 192), 1)
    same_grp = (ri // _K == ci // _K)
    ones_bd = same_grp.astype(jnp.float32)
    tri = (same_grp & (ri >= ci)).astype(jnp.float32)
    s192 = jax.lax.dot(ones_bd, e192, preferred_element_type=jnp.float32)
    t192 = e192 / s192
    cs = jax.lax.dot(tri, t192, preferred_element_type=jnp.float32)
    arow = (-_TAIL + 2.0 * _TAIL * _MBW
            * ((_iota2((192, 1), 0) % _K) + 1).astype(jnp.float32))
    edges = arow + (2.0 * _TAIL * (1.0 - _MBW * _K)) * cs
    lastrow = (_iota2((192, 1), 0) % _K) == _K - 1
    edges = jnp.where(lastrow, _TAIL, edges)
    cw8 = edges[0:96].reshape(_C, _K, _PT)
    ch8 = edges[96:192].reshape(_C, _K, _PT)

    kidx = jax.lax.broadcasted_iota(jnp.int32, (_C, _K, _PT), 1)
    dv = _MD + jax.nn.softplus(jnp.where(kidx == _K - 1, _DCONST, ud8))

    xraw = xz[:, 2 * _W:2 * _W + _PT]
    lane_o = jax.lax.broadcasted_iota(jnp.int32, (1, _PT), 1)
    frozen_o = ((lane_o // _W + lane_o) % 2).astype(jnp.float32)
    xa = xraw * (1.0 - frozen_o)
    xc = jnp.clip(xa, -_TAIL, _TAIL)

    gmat = (_iota2((_C, 96), 0) == _iota2((_C, 96), 1) // _K).astype(jnp.float32)

    def gred(v3):
        return jax.lax.dot(gmat, v3.reshape(96, _PT),
                           preferred_element_type=jnp.float32)

    cmp = (xc[:, None, :] >= cw8).astype(jnp.float32)
    idx = jnp.clip(gred(cmp).astype(jnp.int32), 0, _K - 1)
    ohr = (idx[:, None, :] == kidx).astype(jnp.float32)
    ohl = (idx[:, None, :] == kidx + 1).astype(jnp.float32)
    zat0 = (idx == 0)

    icw = gred(ohl * cw8) + jnp.where(zat0, -_TAIL, 0.0)
    iw = gred(ohr * cw8) - icw
    ich = gred(ohl * ch8) + jnp.where(zat0, -_TAIL, 0.0)
    ih = gred(ohr * ch8) - ich
    id1 = gred(ohr * dv)
    id0 = gred(ohl * dv) + jnp.where(zat0, 1.0, 0.0)
    idl = ih / iw

    theta = (xc - icw) / iw
    t1 = theta * (1.0 - theta)
    th2 = theta * theta
    num = ih * (idl * th2 + id0 * t1)
    den = idl + (id0 + id1 - 2.0 * idl) * t1
    y = ich + num / den
    omt = 1.0 - theta
    dnum = idl * idl * (id1 * th2 + 2.0 * idl * t1 + id0 * omt * omt)
    lad = jnp.log(dnum) - 2.0 * jnp.log(den)

    inside = (xa >= -_TAIL) & (xa <= _TAIL)
    yout = jnp.where(inside, y, xa)
    ladout = jnp.where(inside, lad, 0.0)

    xout_ref[...] = xraw * frozen_o + yout
    lad_ref[...] = jnp.sum(ladout, axis=0, keepdims=True)


def kernel(x, logdet, conditioning, W1, b1, W2, b2, W3, b3):
    xp = jnp.pad(x, ((0, 0), (0, 0), (2, 2), (0, 0))).reshape(_B, _C, (_H + 4) * _W)
    cp = jnp.pad(conditioning, ((0, 0), (0, 0), (2, 2), (0, 0)))
    cp = cp.reshape(_B, _COND, (_H + 4) * _W)

    w1r = jnp.transpose(W1, (2, 3, 0, 1))
    b1r = b1.reshape(_HID, 1)
    w2m = W2[:, :, 0, 0]
    b2r = b2.reshape(_HID, 1)

    perm = np.zeros(_OUTP, np.int32)
    step = 3 * _K - 1
    for c in range(_C):
        for k in range(_K):
            perm[c * _K + k] = c * step + k
            perm[96 + c * _K + k] = c * step + _K + k
        for k in range(7):
            perm[192 + c * _K + k] = c * step + 2 * _K + k
        perm[192 + c * _K + 7] = _C * step
    w3p = jnp.concatenate([W3, jnp.zeros((1,) + W3.shape[1:], W3.dtype)], 0)[perm]
    b3p = jnp.concatenate([b3, jnp.zeros((1,), b3.dtype)], 0)[perm]
    w3r = jnp.transpose(w3p, (2, 3, 0, 1))
    b3r = b3p.reshape(_OUTP, 1)

    xout, ladp = pl.pallas_call(
        _kernel,
        grid=(_B, _NT),
        in_specs=[
            pl.BlockSpec((None, _C, (_H + 4) * _W), lambda b, t: (b, 0, 0)),
            pl.BlockSpec((None, _COND, (_H + 4) * _W), lambda b, t: (b, 0, 0)),
            pl.BlockSpec((3, 3, _HID, _C + _COND), lambda b, t: (0, 0, 0, 0)),
            pl.BlockSpec((_HID, 1), lambda b, t: (0, 0)),
            pl.BlockSpec((_HID, _HID), lambda b, t: (0, 0)),
            pl.BlockSpec((_HID, 1), lambda b, t: (0, 0)),
            pl.BlockSpec((3, 3, _OUTP, _HID), lambda b, t: (0, 0, 0, 0)),
            pl.BlockSpec((_OUTP, 1), lambda b, t: (0, 0)),
        ],
        out_specs=[
            pl.BlockSpec((None, _C, _PT), lambda b, t: (b, 0, t)),
            pl.BlockSpec((None, None, 1, _PT), lambda b, t: (b, t, 0, 0)),
        ],
        out_shape=[
            jax.ShapeDtypeStruct((_B, _C, _H * _W), jnp.float32),
            jax.ShapeDtypeStruct((_B, _NT, 1, _PT), jnp.float32),
        ],
        compiler_params=pltpu.CompilerParams(
            dimension_semantics=("parallel", "arbitrary")),
    )(xp, cp, w1r, b1r, w2m, b2r, w3r, b3r)

    x_out = xout.reshape(_B, _C, _H, _W)
    logdet_out = logdet + jnp.sum(ladp, axis=(1, 2, 3))
    return x_out, logdet_out


# speedup vs baseline: 633.5194x; 1.5366x over previous
import numpy as np
import jax
import jax.numpy as jnp
from jax.experimental import pallas as pl
from jax.experimental.pallas import tpu as pltpu

_K = 8
_TAIL = 3.0
_MBW = 1e-3
_MBH = 1e-3
_MD = 1e-3
_B, _C, _H, _W = 16, 12, 128, 128
_COND = _C // 3
_HID = 64
_OUTP = 288
_TR = 32
_NT = _H // _TR
_PT = _TR * _W
_PH = (_TR + 2) * _W
_PZ = (_TR + 4) * _W
_DCONST = float(np.log(np.expm1(1.0 - _MD)))


def _shift_cols(v, dx, col):
    if dx == 1:
        return v
    if dx == 0:
        u = pltpu.roll(v, 1, axis=1)
        return jnp.where(col == 0, 0.0, u)
    u = pltpu.roll(v, v.shape[1] - 1, axis=1)
    return jnp.where(col == _W - 1, 0.0, u)


def _conv3x3(src, wc, nout_lanes):
    cin, nl = src.shape
    col = jax.lax.broadcasted_iota(jnp.int32, (1, nl), 1) % _W
    acc = None
    for dx in range(3):
        s = _shift_cols(src, dx, col)
        rhs = jnp.concatenate([s[:, dy * _W:dy * _W + nout_lanes]
                               for dy in range(3)], axis=0)
        tap = jax.lax.dot(wc[dx], rhs, preferred_element_type=jnp.float32)
        acc = tap if acc is None else acc + tap
    return acc


def _iota2(shape, dim):
    return jax.lax.broadcasted_iota(jnp.int32, shape, dim)


def _kernel(xp_ref, cp_ref, w1_ref, b1_ref, w2_ref, b2_ref, w3_ref, b3_ref,
            xout_ref, lad_ref):
    t = pl.program_id(1)
    lane_z = jax.lax.broadcasted_iota(jnp.int32, (1, _PZ), 1)
    frozen_z = ((lane_z // _W + lane_z) % 2).astype(jnp.float32)

    xz = xp_ref[:, pl.ds(t * _PT, _PZ)]
    cz = cp_ref[:, pl.ds(t * _PT, _PZ)]
    z = jnp.concatenate([xz * frozen_z, cz], axis=0)

    h1 = jnp.maximum(_conv3x3(z, w1_ref[...], _PH) + b1_ref[...], 0.0)
    h2 = jnp.maximum(
        jax.lax.dot(w2_ref[...], h1, preferred_element_type=jnp.float32)
        + b2_ref[...], 0.0)
    lane_h = jax.lax.broadcasted_iota(jnp.int32, (1, _PH), 1)
    grow = t * _TR - 1 + lane_h // _W
    h2 = jnp.where((grow >= 0) & (grow < _H), h2, 0.0)
    params = _conv3x3(h2, w3_ref[...], _PT) + b3_ref[...]

    uw = params[0:96].reshape(_C, _K, _PT)
    uh = params[96:192].reshape(_C, _K, _PT)
    ud8 = params[192:288].reshape(_C, _K, _PT)

    mw = jnp.max(uw, axis=1, keepdims=True)
    mh = jnp.max(uh, axis=1, keepdims=True)
    e192 = jnp.concatenate([jnp.exp(uw - mw).reshape(96, _PT),
                            jnp.exp(uh - mh).reshape(96, _PT)], axis=0)
    ri = _iota2((192, 192), 0)
    ci = _iota2((192, 192), 1)
    same_grp = (ri // _K == ci // _K)
    ones_bd = same_grp.astype(jnp.float32)
    tri = (same_grp & (ri >= ci)).astype(jnp.float32)
    s192 = jax.lax.dot(ones_bd, e192, preferred_element_type=jnp.float32)
    t192 = e192 / s192
    cs = jax.lax.dot(tri, t192, preferred_element_type=jnp.float32)
    arow = (-_TAIL + 2.0 * _TAIL * _MBW
            * ((_iota2((192, 1), 0) % _K) + 1).astype(jnp.float32))
    edges = arow + (2.0 * _TAIL * (1.0 - _MBW * _K)) * cs
    lastrow = (_iota2((192, 1), 0) % _K) == _K - 1
    edges = jnp.where(lastrow, _TAIL, edges)
    cw8 = edges[0:96].reshape(_C, _K, _PT)
    ch8 = edges[96:192].reshape(_C, _K, _PT)

    kidx = jax.lax.broadcasted_iota(jnp.int32, (_C, _K, _PT), 1)
    dv = _MD + jax.nn.softplus(jnp.where(kidx == _K - 1, _DCONST, ud8))

    xraw = xz[:, 2 * _W:2 * _W + _PT]
    lane_o = jax.lax.broadcasted_iota(jnp.int32, (1, _PT), 1)
    frozen_o = ((lane_o // _W + lane_o) % 2).astype(jnp.float32)
    xa = xraw * (1.0 - frozen_o)
    xc = jnp.clip(xa, -_TAIL, _TAIL)

    gmat = (_iota2((_C, 96), 0) == _iota2((_C, 96), 1) // _K).astype(jnp.float32)

    def gred(v3):
        return jax.lax.dot(gmat, v3.reshape(96, _PT),
                           preferred_element_type=jnp.float32)

    cmp = (xc[:, None, :] >= cw8).astype(jnp.float32)
    idx = jnp.clip(gred(cmp).astype(jnp.int32), 0, _K - 1)
    ohr = (idx[:, None, :] == kidx).astype(jnp.float32)
    ohl = (idx[:, None, :] == kidx + 1).astype(jnp.float32)
    zat0 = (idx == 0)

    icw = gred(ohl * cw8) + jnp.where(zat0, -_TAIL, 0.0)
    iw = gred(ohr * cw8) - icw
    ich = gred(ohl * ch8) + jnp.where(zat0, -_TAIL, 0.0)
    ih = gred(ohr * ch8) - ich
    id1 = gred(ohr * dv)
    id0 = gred(ohl * dv) + jnp.where(zat0, 1.0, 0.0)
    idl = ih / iw

    theta = (xc - icw) / iw
    t1 = theta * (1.0 - theta)
    th2 = theta * theta
    num = ih * (idl * th2 + id0 * t1)
    den = idl + (id0 + id1 - 2.0 * idl) * t1
    y = ich + num / den
    omt = 1.0 - theta
    dnum = idl * idl * (id1 * th2 + 2.0 * idl * t1 + id0 * omt * omt)
    lad = jnp.log(dnum) - 2.0 * jnp.log(den)

    inside = (xa >= -_TAIL) & (xa <= _TAIL)
    yout = jnp.where(inside, y, xa)
    ladout = jnp.where(inside, lad, 0.0)

    xout_ref[...] = xraw * frozen_o + yout
    lad_ref[...] = jnp.sum(ladout, axis=0, keepdims=True)


def kernel(x, logdet, conditioning, W1, b1, W2, b2, W3, b3):
    xp = jnp.pad(x, ((0, 0), (0, 0), (2, 2), (0, 0))).reshape(_B, _C, (_H + 4) * _W)
    cp = jnp.pad(conditioning, ((0, 0), (0, 0), (2, 2), (0, 0)))
    cp = cp.reshape(_B, _COND, (_H + 4) * _W)

    w1c = jnp.stack([jnp.concatenate([W1[:, :, dy, dx] for dy in range(3)],
                                     axis=1) for dx in range(3)])
    b1r = b1.reshape(_HID, 1)
    w2m = W2[:, :, 0, 0]
    b2r = b2.reshape(_HID, 1)

    perm = np.zeros(_OUTP, np.int32)
    step = 3 * _K - 1
    for c in range(_C):
        for k in range(_K):
            perm[c * _K + k] = c * step + k
            perm[96 + c * _K + k] = c * step + _K + k
        for k in range(7):
            perm[192 + c * _K + k] = c * step + 2 * _K + k
        perm[192 + c * _K + 7] = _C * step
    w3p = jnp.concatenate([W3, jnp.zeros((1,) + W3.shape[1:], W3.dtype)], 0)[perm]
    b3p = jnp.concatenate([b3, jnp.zeros((1,), b3.dtype)], 0)[perm]
    w3c = jnp.stack([jnp.concatenate([w3p[:, :, dy, dx] for dy in range(3)],
                                     axis=1) for dx in range(3)])
    b3r = b3p.reshape(_OUTP, 1)

    xout, ladp = pl.pallas_call(
        _kernel,
        grid=(_B, _NT),
        in_specs=[
            pl.BlockSpec((None, _C, (_H + 4) * _W), lambda b, t: (b, 0, 0)),
            pl.BlockSpec((None, _COND, (_H + 4) * _W), lambda b, t: (b, 0, 0)),
            pl.BlockSpec((3, _HID, 3 * (_C + _COND)), lambda b, t: (0, 0, 0)),
            pl.BlockSpec((_HID, 1), lambda b, t: (0, 0)),
            pl.BlockSpec((_HID, _HID), lambda b, t: (0, 0)),
            pl.BlockSpec((_HID, 1), lambda b, t: (0, 0)),
            pl.BlockSpec((3, _OUTP, 3 * _HID), lambda b, t: (0, 0, 0)),
            pl.BlockSpec((_OUTP, 1), lambda b, t: (0, 0)),
        ],
        out_specs=[
            pl.BlockSpec((None, _C, _PT), lambda b, t: (b, 0, t)),
            pl.BlockSpec((None, None, 1, _PT), lambda b, t: (b, t, 0, 0)),
        ],
        out_shape=[
            jax.ShapeDtypeStruct((_B, _C, _H * _W), jnp.float32),
            jax.ShapeDtypeStruct((_B, _NT, 1, _PT), jnp.float32),
        ],
        compiler_params=pltpu.CompilerParams(
            dimension_semantics=("parallel", "arbitrary")),
    )(xp, cp, w1c, b1r, w2m, b2r, w3c, b3r)

    x_out = xout.reshape(_B, _C, _H, _W)
    logdet_out = logdet + jnp.sum(ladp, axis=(1, 2, 3))
    return x_out, logdet_out
